# Initial kernel scaffold; baseline (speedup 1.0000x reference)
import jax
import jax.numpy as jnp
from jax.experimental import pallas as pl


def kernel(sampleU, sampleI, sampleJ, sampleR, alpha, betaU, betaI, gammaUI, gammaIU, gammaIJ, gammaJI):
    raise NotImplementedError("write your pallas kernel here")


# speedup vs baseline: 9.2308x; 9.2308x over previous
import functools

import jax
import jax.numpy as jnp
from jax import lax
from jax.experimental import pallas as pl
from jax.experimental.pallas import tpu as pltpu
from jax.experimental.pallas import tpu_sc as plsc

_N = 1000
_NP = 1024
_K = 128
_B = 16384
_NC, _NS, _L = 2, 16, 16
_NW = _NC * _NS
_BPW = _B // _NW
_ROWS = _BPW // 128


def _gram_body(a_ref, b_ref, br_ref, bc_ref, o_ref):
    acc = lax.dot_general(
        a_ref[0], b_ref[0],
        dimension_numbers=(((1,), (1,)), ((), ())),
        preferred_element_type=jnp.float32,
    )
    o_ref[0] = acc + br_ref[0, 0][:, None] + bc_ref[0, 0][None, :]


def _gram(a, b, br, bc):
    return pl.pallas_call(
        _gram_body,
        grid=(2,),
        in_specs=[
            pl.BlockSpec((1, _NP, _K), lambda t: (t, 0, 0)),
            pl.BlockSpec((1, _NP, _K), lambda t: (t, 0, 0)),
            pl.BlockSpec((1, 1, _NP), lambda t: (t, 0, 0)),
            pl.BlockSpec((1, 1, _NP), lambda t: (t, 0, 0)),
        ],
        out_specs=pl.BlockSpec((1, _NP, _NP), lambda t: (t, 0, 0)),
        out_shape=jax.ShapeDtypeStruct((2, _NP, _NP), jnp.float32),
    )(a, b, br, bc)


_MESH = plsc.VectorSubcoreMesh(
    core_axis_name="c", subcore_axis_name="s",
    num_cores=_NC, num_subcores=_NS,
)


@functools.partial(
    pl.kernel,
    out_type=jax.ShapeDtypeStruct((_NW, _L), jnp.float32),
    mesh=_MESH,
    scratch_types=[
        pltpu.VMEM((_ROWS, 128), jnp.int32),
        pltpu.VMEM((_ROWS, 128), jnp.int32),
        pltpu.VMEM((_ROWS, 128), jnp.int32),
        pltpu.VMEM((_ROWS, 128), jnp.float32),
        pltpu.VMEM((_ROWS, 128), jnp.int32),
        pltpu.VMEM((_ROWS, 128), jnp.int32),
        pltpu.VMEM((_ROWS, 128), jnp.float32),
        pltpu.VMEM((_ROWS, 128), jnp.float32),
        pltpu.VMEM((_L,), jnp.float32),
        pltpu.VMEM((_L,), jnp.float32),
        pltpu.SemaphoreType.DMA,
    ],
)
def _sc_loss(u_hbm, i_hbm, j_hbm, r_hbm, alpha_hbm, gh_hbm, out_hbm,
             u_v, i_v, j_v, r_v, fui_v, fij_v, g_v, h_v, alpha_v, acc_v,
             sem):
    wid = lax.axis_index("s") * _NC + lax.axis_index("c")
    base = wid * _ROWS
    pltpu.sync_copy(u_hbm.at[pl.ds(base, _ROWS)], u_v)
    pltpu.sync_copy(i_hbm.at[pl.ds(base, _ROWS)], i_v)
    pltpu.sync_copy(j_hbm.at[pl.ds(base, _ROWS)], j_v)
    pltpu.sync_copy(r_hbm.at[pl.ds(base, _ROWS)], r_v)
    pltpu.sync_copy(alpha_hbm, alpha_v)

    for g in range(_ROWS):
        for k in range(128 // _L):
            s = k * _L
            uu = u_v[g, pl.ds(s, _L)]
            ii = i_v[g, pl.ds(s, _L)]
            jj = j_v[g, pl.ds(s, _L)]
            fui_v[g, pl.ds(s, _L)] = uu * _NP + ii
            fij_v[g, pl.ds(s, _L)] = ii * _NP + jj + _NP * _NP

    copies = []
    for g in range(_ROWS):
        copies.append(pltpu.async_copy(gh_hbm.at[fui_v.at[g]], g_v.at[g], sem))
        copies.append(pltpu.async_copy(gh_hbm.at[fij_v.at[g]], h_v.at[g], sem))
    for c in copies:
        c.wait()

    alpha = alpha_v[...]
    acc = jnp.zeros((_L,), jnp.float32)
    for g in range(_ROWS):
        for k in range(128 // _L):
            s = k * _L
            d = alpha + g_v[g, pl.ds(s, _L)] + h_v[g, pl.ds(s, _L)] \
                - r_v[g, pl.ds(s, _L)]
            acc = acc + d * d
    acc_v[...] = acc
    pltpu.sync_copy(acc_v, out_hbm.at[wid])


def kernel(sampleU, sampleI, sampleJ, sampleR, alpha, betaU, betaI,
           gammaUI, gammaIU, gammaIJ, gammaJI):
    pad_m = ((0, 0), (0, _NP - _N), (0, 0))
    a = jnp.pad(jnp.stack([gammaUI, gammaIJ]), pad_m)
    b = jnp.pad(jnp.stack([gammaIU, gammaJI]), pad_m)
    zeros = jnp.zeros_like(betaU)
    br = jnp.pad(jnp.stack([betaU, zeros]),
                 ((0, 0), (0, _NP - _N))).reshape(2, 1, _NP)
    bc = jnp.pad(jnp.stack([betaI, zeros]),
                 ((0, 0), (0, _NP - _N))).reshape(2, 1, _NP)
    gh = _gram(a, b, br, bc).reshape(-1)

    u2 = sampleU.reshape(_B // 128, 128)
    i2 = sampleI.reshape(_B // 128, 128)
    j2 = sampleJ.reshape(_B // 128, 128)
    r2 = sampleR.reshape(_B // 128, 128)
    alpha16 = jnp.full((_L,), alpha, jnp.float32)

    parts = _sc_loss(u2, i2, j2, r2, alpha16, gh)
    return jnp.sum(parts) * (0.5 / _B)


# speedup vs baseline: 9.5708x; 1.0368x over previous
import functools

import jax
import jax.numpy as jnp
from jax import lax
from jax.experimental import pallas as pl
from jax.experimental.pallas import tpu as pltpu
from jax.experimental.pallas import tpu_sc as plsc

_N = 1000
_NP = 1024
_K = 128
_B = 16384
_NC, _NS, _L = 2, 16, 16
_NW = _NC * _NS
_BPW = _B // _NW
_ROWS = _BPW // 128


def _gram_body(a_ref, b_ref, br_ref, bc_ref, o_ref):
    acc = lax.dot_general(
        a_ref[0], b_ref[0, 0],
        dimension_numbers=(((1,), (1,)), ((), ())),
        preferred_element_type=jnp.float32,
    )
    o_ref[0, 0] = acc + br_ref[0, :, 0][:, None] + bc_ref[0, 0, 0][None, :]


def _gram(a, b, br, bc):
    return pl.pallas_call(
        _gram_body,
        grid=(2, 8),
        in_specs=[
            pl.BlockSpec((1, _NP, _K), lambda t, p: (t, 0, 0)),
            pl.BlockSpec((1, 1, _K, _K), lambda t, p: (t, p, 0, 0)),
            pl.BlockSpec((1, _NP, 1), lambda t, p: (t, 0, 0)),
            pl.BlockSpec((1, 1, 1, _K), lambda t, p: (t, p, 0, 0)),
        ],
        out_specs=pl.BlockSpec((1, 1, _NP, _K), lambda t, p: (t, p, 0, 0)),
        out_shape=jax.ShapeDtypeStruct((2, 8, _NP, _K), jnp.float32),
    )(a, b, br, bc)


_MESH = plsc.VectorSubcoreMesh(
    core_axis_name="c", subcore_axis_name="s",
    num_cores=_NC, num_subcores=_NS,
)


@functools.partial(
    pl.kernel,
    out_type=jax.ShapeDtypeStruct((_NW, _L), jnp.float32),
    mesh=_MESH,
    scratch_types=[
        pltpu.VMEM((_BPW,), jnp.int32),
        pltpu.VMEM((_BPW,), jnp.int32),
        pltpu.VMEM((_BPW,), jnp.int32),
        pltpu.VMEM((_BPW,), jnp.float32),
        pltpu.VMEM((_ROWS, 128), jnp.int32),
        pltpu.VMEM((_ROWS, 128), jnp.int32),
        pltpu.VMEM((_ROWS, 128), jnp.float32),
        pltpu.VMEM((_ROWS, 128), jnp.float32),
        pltpu.VMEM((_L,), jnp.float32),
        pltpu.SemaphoreType.DMA,
    ],
)
def _sc_loss(u_hbm, i_hbm, j_hbm, r_hbm, gh_hbm, out_hbm,
             u_v, i_v, j_v, r_v, fui_v, fij_v, g_v, h_v, acc_v, sem):
    wid = lax.axis_index("s") * _NC + lax.axis_index("c")
    base = wid * _BPW
    pltpu.sync_copy(u_hbm.at[pl.ds(base, _BPW)], u_v)
    pltpu.sync_copy(i_hbm.at[pl.ds(base, _BPW)], i_v)
    pltpu.sync_copy(j_hbm.at[pl.ds(base, _BPW)], j_v)
    pltpu.sync_copy(r_hbm.at[pl.ds(base, _BPW)], r_v)

    for g in range(_ROWS):
        for k in range(128 // _L):
            s = g * 128 + k * _L
            uu = u_v[pl.ds(s, _L)]
            ii = i_v[pl.ds(s, _L)]
            jj = j_v[pl.ds(s, _L)]
            fui_v[g, pl.ds(k * _L, _L)] = (
                (ii >> 7) * 131072 + uu * 128 + (ii & 127))
            fij_v[g, pl.ds(k * _L, _L)] = (
                1048576 + (jj >> 7) * 131072 + ii * 128 + (jj & 127))

    copies = []
    for g in range(_ROWS):
        copies.append(pltpu.async_copy(gh_hbm.at[fui_v.at[g]], g_v.at[g], sem))
        copies.append(pltpu.async_copy(gh_hbm.at[fij_v.at[g]], h_v.at[g], sem))
    for c in copies:
        c.wait()

    acc = jnp.zeros((_L,), jnp.float32)
    for g in range(_ROWS):
        for k in range(128 // _L):
            s = k * _L
            d = g_v[g, pl.ds(s, _L)] + h_v[g, pl.ds(s, _L)] \
                - r_v[pl.ds(g * 128 + s, _L)]
            acc = acc + d * d
    acc_v[...] = acc
    pltpu.sync_copy(acc_v, out_hbm.at[wid])


def kernel(sampleU, sampleI, sampleJ, sampleR, alpha, betaU, betaI,
           gammaUI, gammaIU, gammaIJ, gammaJI):
    pad_m = ((0, 0), (0, _NP - _N), (0, 0))
    a = jnp.pad(jnp.stack([gammaUI, gammaIJ]), pad_m)
    b = jnp.pad(jnp.stack([gammaIU, gammaJI]), pad_m).reshape(2, 8, _K, _K)
    zeros = jnp.zeros_like(betaU)
    br = jnp.pad(jnp.stack([betaU, zeros]),
                 ((0, 0), (0, _NP - _N))).reshape(2, _NP, 1)
    bc = jnp.pad(jnp.stack([betaI + alpha, zeros]),
                 ((0, 0), (0, _NP - _N))).reshape(2, 8, 1, _K)
    gh = _gram(a, b, br, bc).reshape(-1)

    parts = _sc_loss(sampleU, sampleI, sampleJ, sampleR, gh)
    return jnp.sum(parts) * (0.5 / _B)


# speedup vs baseline: 13.7242x; 1.4340x over previous
import functools

import jax
import jax.numpy as jnp
from jax import lax
from jax.experimental import pallas as pl
from jax.experimental.pallas import tpu as pltpu
from jax.experimental.pallas import tpu_sc as plsc

_N = 1000
_K = 128
_B = 16384
_NC, _NS, _L = 2, 16, 16
_NW = _NC * _NS
_BPW = _B // _NW
_ROWS = _BPW // 128
_PSTRIDE = _N * _K
_TSTRIDE = 8 * _PSTRIDE


def _gram_body(ui_ref, iu_ref, ij_ref, ji_ref, bu_ref, bi_ref, o_ref):
    dn = (((1,), (1,)), ((), ()))
    accg = lax.dot_general(ui_ref[...], iu_ref[...], dn,
                           preferred_element_type=jnp.float32)
    accg = accg + bu_ref[0][:, None] + bi_ref[0][None, :]
    accg = jnp.pad(accg, ((0, 0), (0, 24)))
    acch = lax.dot_general(ij_ref[...], ji_ref[...], dn,
                           preferred_element_type=jnp.float32)
    acch = jnp.pad(acch, ((0, 0), (0, 24)))
    for p in range(8):
        o_ref[0, p] = accg[:, 128 * p:128 * (p + 1)]
        o_ref[1, p] = acch[:, 128 * p:128 * (p + 1)]


def _gram(ui, iu, ij, ji, bu, bi):
    return pl.pallas_call(
        _gram_body,
        out_shape=jax.ShapeDtypeStruct((2, 8, _N, _K), jnp.float32),
    )(ui, iu, ij, ji, bu, bi)


_MESH = plsc.VectorSubcoreMesh(
    core_axis_name="c", subcore_axis_name="s",
    num_cores=_NC, num_subcores=_NS,
)


@functools.partial(
    pl.kernel,
    out_type=jax.ShapeDtypeStruct((_NW, _L), jnp.float32),
    mesh=_MESH,
    scratch_types=[
        pltpu.VMEM((_BPW,), jnp.int32),
        pltpu.VMEM((_BPW,), jnp.int32),
        pltpu.VMEM((_BPW,), jnp.int32),
        pltpu.VMEM((_BPW,), jnp.float32),
        pltpu.VMEM((_ROWS, 128), jnp.int32),
        pltpu.VMEM((_ROWS, 128), jnp.int32),
        pltpu.VMEM((_ROWS, 128), jnp.float32),
        pltpu.VMEM((_ROWS, 128), jnp.float32),
        pltpu.VMEM((_L,), jnp.float32),
        pltpu.SemaphoreType.DMA,
    ],
)
def _sc_loss(u_hbm, i_hbm, j_hbm, r_hbm, gh_hbm, out_hbm,
             u_v, i_v, j_v, r_v, fui_v, fij_v, g_v, h_v, acc_v, sem):
    wid = lax.axis_index("s") * _NC + lax.axis_index("c")
    base = wid * _BPW
    pltpu.sync_copy(u_hbm.at[pl.ds(base, _BPW)], u_v)
    pltpu.sync_copy(i_hbm.at[pl.ds(base, _BPW)], i_v)
    pltpu.sync_copy(j_hbm.at[pl.ds(base, _BPW)], j_v)
    pltpu.sync_copy(r_hbm.at[pl.ds(base, _BPW)], r_v)

    for g in range(_ROWS):
        for k in range(128 // _L):
            s = g * 128 + k * _L
            uu = u_v[pl.ds(s, _L)]
            ii = i_v[pl.ds(s, _L)]
            jj = j_v[pl.ds(s, _L)]
            fui_v[g, pl.ds(k * _L, _L)] = (
                (ii >> 7) * _PSTRIDE + uu * 128 + (ii & 127))
            fij_v[g, pl.ds(k * _L, _L)] = (
                _TSTRIDE + (jj >> 7) * _PSTRIDE + ii * 128 + (jj & 127))

    copies = []
    for g in range(_ROWS):
        copies.append(pltpu.async_copy(gh_hbm.at[fui_v.at[g]], g_v.at[g], sem))
        copies.append(pltpu.async_copy(gh_hbm.at[fij_v.at[g]], h_v.at[g], sem))
    for c in copies:
        c.wait()

    acc = jnp.zeros((_L,), jnp.float32)
    for g in range(_ROWS):
        for k in range(128 // _L):
            s = k * _L
            d = g_v[g, pl.ds(s, _L)] + h_v[g, pl.ds(s, _L)] \
                - r_v[pl.ds(g * 128 + s, _L)]
            acc = acc + d * d
    acc_v[...] = acc
    pltpu.sync_copy(acc_v, out_hbm.at[wid])


def kernel(sampleU, sampleI, sampleJ, sampleR, alpha, betaU, betaI,
           gammaUI, gammaIU, gammaIJ, gammaJI):
    bu = betaU.reshape(1, _N)
    bi = (betaI + alpha).reshape(1, _N)
    gh = _gram(gammaUI, gammaIU, gammaIJ, gammaJI, bu, bi).reshape(-1)
    parts = _sc_loss(sampleU, sampleI, sampleJ, sampleR, gh)
    return jnp.sum(parts) * (0.5 / _B)
